# Initial kernel scaffold; baseline (speedup 1.0000x reference)
import jax
import jax.numpy as jnp
from jax.experimental import pallas as pl


def kernel(feats, feats_deep, rna_seq, params):
    raise NotImplementedError("write your pallas kernel here")


# speedup vs baseline: 2.2305x; 2.2305x over previous
import functools

import jax
import jax.numpy as jnp
from jax import lax
from jax.experimental import pallas as pl
from jax.experimental.pallas import tpu as pltpu

B_ = 4
N_ = 8192
D_ = 512
DD_ = 256
K_ = 10
NRNA_ = 4848
MIXL_ = 4

NB_ = 16
NBLK_ = N_ // NB_


def _ln(x, g, beta):
    m = x.mean(-1, keepdims=True)
    v = ((x - m) ** 2).mean(-1, keepdims=True)
    return (x - m) * lax.rsqrt(v + 1e-5) * g + beta


def _gelu(x):
    return x * 0.5 * (1.0 + lax.erf(x * 0.7071067811865476))


def _attn_body(fd_ref, p1w, p1b, g1, be1, p2w, p2b, g2, be2,
               aw, ab, bw, bb, cw, cb,
               logits_ref, m_ref, s_ref, bds_ref,
               m_sc, s_sc, w_sc):
    nb = pl.program_id(1)

    @pl.when(nb == 0)
    def _init():
        m_sc[0, 0] = -1e30
        s_sc[0, 0] = 0.0
        w_sc[...] = jnp.zeros_like(w_sc)

    fd = fd_ref[0]
    x = _gelu(_ln(fd @ p1w[...] + p1b[...], g1[...], be1[...]))
    v = _gelu(_ln(x @ p2w[...] + p2b[...], g2[...], be2[...]))
    a = jnp.tanh(v @ aw[...] + ab[...])
    sgt = jax.nn.sigmoid(v @ bw[...] + bb[...])
    gate = a * sgt
    logit = jnp.sum(gate * cw[...], axis=1, keepdims=True) + cb[...]
    logits_ref[0, 0] = logit

    m_old = m_sc[0, 0]
    mb = jnp.max(logit)
    m_new = jnp.maximum(m_old, mb)
    alpha = jnp.exp(m_old - m_new)
    e = jnp.exp(logit - m_new)
    s_new = s_sc[0, 0] * alpha + jnp.sum(e)
    wv = lax.dot_general(e, v, (((0,), (0,)), ((), ())),
                         preferred_element_type=jnp.float32)
    w_sc[...] = w_sc[...] * alpha + wv
    m_sc[0, 0] = m_new
    s_sc[0, 0] = s_new

    @pl.when(nb == NB_ - 1)
    def _fin():
        m_ref[...] = jnp.reshape(m_new, (1, 1, 1))
        s_ref[...] = jnp.reshape(s_new, (1, 1, 1))
        bds_ref[0] = w_sc[...] / s_new


def _attn_call(feats_deep, P):
    pre1, pre2 = P["pre1"], P["pre2"]
    ga = P["attn_deep"]
    args = (feats_deep,
            pre1["W"], pre1["b"].reshape(1, DD_),
            P["preln1"]["g"].reshape(1, DD_), P["preln1"]["beta"].reshape(1, DD_),
            pre2["W"], pre2["b"].reshape(1, DD_),
            P["preln2"]["g"].reshape(1, DD_), P["preln2"]["beta"].reshape(1, DD_),
            ga["a"]["W"], ga["a"]["b"].reshape(1, DD_),
            ga["b"]["W"], ga["b"]["b"].reshape(1, DD_),
            ga["c"]["W"].reshape(1, DD_), ga["c"]["b"].reshape(1, 1))
    full = lambda s: pl.BlockSpec(s, lambda b, nb: (0,) * len(s))
    out = pl.pallas_call(
        _attn_body,
        grid=(B_, NB_),
        in_specs=[
            pl.BlockSpec((1, NBLK_, DD_), lambda b, nb: (b, nb, 0)),
            full((DD_, DD_)), full((1, DD_)), full((1, DD_)), full((1, DD_)),
            full((DD_, DD_)), full((1, DD_)), full((1, DD_)), full((1, DD_)),
            full((DD_, DD_)), full((1, DD_)),
            full((DD_, DD_)), full((1, DD_)),
            full((1, DD_)), full((1, 1)),
        ],
        out_specs=[
            pl.BlockSpec((1, 1, NBLK_, 1), lambda b, nb: (b, nb, 0, 0)),
            pl.BlockSpec((1, 1, 1), lambda b, nb: (b, 0, 0)),
            pl.BlockSpec((1, 1, 1), lambda b, nb: (b, 0, 0)),
            pl.BlockSpec((1, 1, DD_), lambda b, nb: (b, 0, 0)),
        ],
        out_shape=[
            jax.ShapeDtypeStruct((B_, NB_, NBLK_, 1), jnp.float32),
            jax.ShapeDtypeStruct((B_, 1, 1), jnp.float32),
            jax.ShapeDtypeStruct((B_, 1, 1), jnp.float32),
            jax.ShapeDtypeStruct((B_, 1, DD_), jnp.float32),
        ],
        scratch_shapes=[
            pltpu.SMEM((1, 1), jnp.float32),
            pltpu.SMEM((1, 1), jnp.float32),
            pltpu.VMEM((1, DD_), jnp.float32),
        ],
    )(*args)
    logits, m, s, bds = out
    return (logits.reshape(B_, N_), m.reshape(B_, 1), s.reshape(B_, 1),
            bds.reshape(B_, DD_))


def _finalize_body(logits_ref, m_ref, s_ref, bds_ref, cw_ref, cb_ref,
                   ap_ref, proj_ref):
    ap_ref[...] = jnp.exp(logits_ref[...] - m_ref[...]) / s_ref[...]
    proj_ref[...] = bds_ref[...] @ cw_ref[...] + cb_ref[...]


def _finalize_call(logits, m, s, bds, P):
    return pl.pallas_call(
        _finalize_body,
        out_shape=[
            jax.ShapeDtypeStruct((B_, N_), jnp.float32),
            jax.ShapeDtypeStruct((B_, D_), jnp.float32),
        ],
    )(logits, m, s, bds, P["clip"]["W"], P["clip"]["b"].reshape(1, D_))


def _mixer_body(*refs):
    tk_ref = refs[0]
    mx = refs[1:1 + 8 * MIXL_]
    aaw, aab, abw, abb, acw, acb = refs[1 + 8 * MIXL_:1 + 8 * MIXL_ + 6]
    bsel_ref, psum_ref, af_ref = refs[-3:]

    u = tk_ref[0]
    for l in range(MIXL_):
        t1w, t1b, t2w, t2b, c1w, c1b, c2w, c2b = mx[8 * l:8 * l + 8]
        t = _gelu(u @ t1w[...] + t1b[...]) @ t2w[...] + t2b[...]
        u = u + t
        h = _gelu(lax.dot_general(u, c1w[...], (((0,), (0,)), ((), ())),
                                  preferred_element_type=jnp.float32)
                  + c1b[...])
        ct = lax.dot_general(c2w[...], h, (((0,), (1,)), ((), ())),
                             preferred_element_type=jnp.float32)
        u = u + ct + c2b[...].reshape(K_, 1)

    a2 = jnp.tanh(lax.dot_general(aaw[...], u, (((0,), (0,)), ((), ())),
                                  preferred_element_type=jnp.float32)
                  + aab[...].reshape(DD_, 1))
    s2 = jax.nn.sigmoid(lax.dot_general(abw[...], u, (((0,), (0,)), ((), ())),
                                        preferred_element_type=jnp.float32)
                        + abb[...].reshape(DD_, 1))
    af = jnp.sum(a2 * s2 * acw[...].reshape(DD_, 1), axis=0, keepdims=True)
    af = jax.nn.sigmoid(af + acb[...])
    psum = jnp.sum(tk_ref[0], axis=0, keepdims=True)
    af_ref[0] = af
    psum_ref[0] = psum
    bsel_ref[0] = psum * af


def _mixer_call(topk_feats, P):
    args = [topk_feats]
    for lp in P["mixer"]:
        args += [lp["t1"]["W"], lp["t1"]["b"].reshape(1, DD_),
                 lp["t2"]["W"], lp["t2"]["b"].reshape(1, D_),
                 lp["c1"]["W"], lp["c1"]["b"].reshape(1, DD_),
                 lp["c2"]["W"], lp["c2"]["b"].reshape(1, K_)]
    ga = P["aux_ga"]
    args += [ga["a"]["W"], ga["a"]["b"],
             ga["b"]["W"], ga["b"]["b"],
             ga["c"]["W"].reshape(1, DD_), ga["c"]["b"].reshape(1, 1)]

    in_specs = [pl.BlockSpec((1, K_, D_), lambda b: (b, 0, 0))]
    for a in args[1:]:
        s = a.shape
        in_specs.append(pl.BlockSpec(s, lambda b, s=s: (0,) * len(s)))
    row = pl.BlockSpec((1, 1, D_), lambda b: (b, 0, 0))
    out = pl.pallas_call(
        _mixer_body,
        grid=(B_,),
        in_specs=in_specs,
        out_specs=[row, row, row],
        out_shape=[jax.ShapeDtypeStruct((B_, 1, D_), jnp.float32)] * 3,
    )(*args)
    return tuple(o.reshape(B_, D_) for o in out)


RNA_RB_ = 512
RNA_NRB_ = (NRNA_ + RNA_RB_ - 1) // RNA_RB_


def _rna_layer_body(x_ref, w_ref, b_ref, g_ref, be_ref, o_ref, acc):
    rb = pl.program_id(0)

    @pl.when(rb == 0)
    def _init():
        acc[...] = jnp.zeros_like(acc)

    bound = NRNA_ - rb * RNA_RB_
    xb = x_ref[...]
    wb = w_ref[...]
    lane = lax.broadcasted_iota(jnp.int32, (1, RNA_RB_), 1)
    sub = lax.broadcasted_iota(jnp.int32, (RNA_RB_, 1), 0)
    xb = jnp.where(lane < bound, xb, 0.0)
    wb = jnp.where(sub < bound, wb, 0.0)
    acc[...] += lax.dot_general(xb, wb, (((1,), (0,)), ((), ())),
                                preferred_element_type=jnp.float32)

    @pl.when(rb == RNA_NRB_ - 1)
    def _fin():
        y = acc[...] + b_ref[...]
        o_ref[...] = jnp.maximum(_ln(y, g_ref[...], be_ref[...]), 0.0)


def _rna_layer_call(x, W, b, g, beta):
    return pl.pallas_call(
        _rna_layer_body,
        grid=(RNA_NRB_,),
        in_specs=[
            pl.BlockSpec((B_, RNA_RB_), lambda rb: (0, rb)),
            pl.BlockSpec((RNA_RB_, NRNA_), lambda rb: (rb, 0)),
            pl.BlockSpec((1, NRNA_), lambda rb: (0, 0)),
            pl.BlockSpec((1, NRNA_), lambda rb: (0, 0)),
            pl.BlockSpec((1, NRNA_), lambda rb: (0, 0)),
        ],
        out_specs=pl.BlockSpec((B_, NRNA_), lambda rb: (0, 0)),
        out_shape=jax.ShapeDtypeStruct((B_, NRNA_), jnp.float32),
        scratch_shapes=[pltpu.VMEM((B_, NRNA_), jnp.float32)],
    )(x, W, b.reshape(1, NRNA_), g.reshape(1, NRNA_), beta.reshape(1, NRNA_))


def _rna_out_body(r_ref, w_ref, b_ref, cw_ref, cb_ref, emb_ref, proj_ref):
    emb = r_ref[...] @ w_ref[...] + b_ref[...]
    emb_ref[...] = emb
    proj_ref[...] = emb @ cw_ref[...] + cb_ref[...]


def _rna_out_call(r, P):
    return pl.pallas_call(
        _rna_out_body,
        out_shape=[
            jax.ShapeDtypeStruct((B_, DD_), jnp.float32),
            jax.ShapeDtypeStruct((B_, D_), jnp.float32),
        ],
    )(r, P["rna_out"]["W"], P["rna_out"]["b"].reshape(1, DD_),
      P["clip_rna"]["W"], P["clip_rna"]["b"].reshape(1, D_))


def _topk_gather(logits, feats):
    _, idx = lax.top_k(logits, K_)
    idxf = jnp.broadcast_to(idx[:, :, None], (B_, K_, D_))
    return jnp.take_along_axis(feats, idxf, axis=1)


def kernel(feats, feats_deep, rna_seq, params):
    P = params
    logits, m, s, bds = _attn_call(feats_deep, P)
    a_patch, bdp = _finalize_call(logits, m, s, bds, P)
    topk_feats = _topk_gather(logits, feats)
    bsel, psum, a_feat = _mixer_call(topk_feats, P)
    r = _rna_layer_call(rna_seq, P["rna1"]["W"], P["rna1"]["b"],
                        P["rnaln1"]["g"], P["rnaln1"]["beta"])
    r = _rna_layer_call(r, P["rna2"]["W"], P["rna2"]["b"],
                        P["rnaln2"]["g"], P["rnaln2"]["beta"])
    rna_emb, rna_proj = _rna_out_call(r, P)
    return (bsel, psum, bdp, bds, rna_proj, rna_emb, a_feat,
            a_patch.reshape(B_, N_, 1))


# speedup vs baseline: 2.2307x; 1.0001x over previous
import functools

import jax
import jax.numpy as jnp
from jax import lax
from jax.experimental import pallas as pl
from jax.experimental.pallas import tpu as pltpu
from jax.experimental.pallas import tpu_sc as plsc

B_ = 4
N_ = 8192
D_ = 512
DD_ = 256
K_ = 10
NRNA_ = 4848
MIXL_ = 4

NB_ = 16
NBLK_ = N_ // NB_


def _ln(x, g, beta):
    m = x.mean(-1, keepdims=True)
    v = ((x - m) ** 2).mean(-1, keepdims=True)
    return (x - m) * lax.rsqrt(v + 1e-5) * g + beta


def _gelu(x):
    return x * 0.5 * (1.0 + lax.erf(x * 0.7071067811865476))


def _attn_body(fd_ref, p1w, p1b, g1, be1, p2w, p2b, g2, be2,
               aw, ab, bw, bb, cw, cb,
               logits_ref, m_ref, s_ref, bds_ref,
               m_sc, s_sc, w_sc):
    nb = pl.program_id(1)

    @pl.when(nb == 0)
    def _init():
        m_sc[0, 0] = -1e30
        s_sc[0, 0] = 0.0
        w_sc[...] = jnp.zeros_like(w_sc)

    fd = fd_ref[0]
    x = _gelu(_ln(fd @ p1w[...] + p1b[...], g1[...], be1[...]))
    v = _gelu(_ln(x @ p2w[...] + p2b[...], g2[...], be2[...]))
    a = jnp.tanh(v @ aw[...] + ab[...])
    sgt = jax.nn.sigmoid(v @ bw[...] + bb[...])
    gate = a * sgt
    logit = jnp.sum(gate * cw[...], axis=1, keepdims=True) + cb[...]
    logits_ref[0, 0] = logit

    m_old = m_sc[0, 0]
    mb = jnp.max(logit)
    m_new = jnp.maximum(m_old, mb)
    alpha = jnp.exp(m_old - m_new)
    e = jnp.exp(logit - m_new)
    s_new = s_sc[0, 0] * alpha + jnp.sum(e)
    wv = lax.dot_general(e, v, (((0,), (0,)), ((), ())),
                         preferred_element_type=jnp.float32)
    w_sc[...] = w_sc[...] * alpha + wv
    m_sc[0, 0] = m_new
    s_sc[0, 0] = s_new

    @pl.when(nb == NB_ - 1)
    def _fin():
        m_ref[...] = jnp.reshape(m_new, (1, 1, 1))
        s_ref[...] = jnp.reshape(s_new, (1, 1, 1))
        bds_ref[0] = w_sc[...] / s_new


def _attn_call(feats_deep, P):
    pre1, pre2 = P["pre1"], P["pre2"]
    ga = P["attn_deep"]
    args = (feats_deep,
            pre1["W"], pre1["b"].reshape(1, DD_),
            P["preln1"]["g"].reshape(1, DD_), P["preln1"]["beta"].reshape(1, DD_),
            pre2["W"], pre2["b"].reshape(1, DD_),
            P["preln2"]["g"].reshape(1, DD_), P["preln2"]["beta"].reshape(1, DD_),
            ga["a"]["W"], ga["a"]["b"].reshape(1, DD_),
            ga["b"]["W"], ga["b"]["b"].reshape(1, DD_),
            ga["c"]["W"].reshape(1, DD_), ga["c"]["b"].reshape(1, 1))
    full = lambda s: pl.BlockSpec(s, lambda b, nb: (0,) * len(s))
    out = pl.pallas_call(
        _attn_body,
        grid=(B_, NB_),
        in_specs=[
            pl.BlockSpec((1, NBLK_, DD_), lambda b, nb: (b, nb, 0)),
            full((DD_, DD_)), full((1, DD_)), full((1, DD_)), full((1, DD_)),
            full((DD_, DD_)), full((1, DD_)), full((1, DD_)), full((1, DD_)),
            full((DD_, DD_)), full((1, DD_)),
            full((DD_, DD_)), full((1, DD_)),
            full((1, DD_)), full((1, 1)),
        ],
        out_specs=[
            pl.BlockSpec((1, 1, NBLK_, 1), lambda b, nb: (b, nb, 0, 0)),
            pl.BlockSpec((1, 1, 1), lambda b, nb: (b, 0, 0)),
            pl.BlockSpec((1, 1, 1), lambda b, nb: (b, 0, 0)),
            pl.BlockSpec((1, 1, DD_), lambda b, nb: (b, 0, 0)),
        ],
        out_shape=[
            jax.ShapeDtypeStruct((B_, NB_, NBLK_, 1), jnp.float32),
            jax.ShapeDtypeStruct((B_, 1, 1), jnp.float32),
            jax.ShapeDtypeStruct((B_, 1, 1), jnp.float32),
            jax.ShapeDtypeStruct((B_, 1, DD_), jnp.float32),
        ],
        scratch_shapes=[
            pltpu.SMEM((1, 1), jnp.float32),
            pltpu.SMEM((1, 1), jnp.float32),
            pltpu.VMEM((1, DD_), jnp.float32),
        ],
    )(*args)
    logits, m, s, bds = out
    return (logits.reshape(B_, N_), m.reshape(B_, 1), s.reshape(B_, 1),
            bds.reshape(B_, DD_))


def _finalize_body(logits_ref, m_ref, s_ref, bds_ref, cw_ref, cb_ref,
                   cv_ref, ci_ref, ap_ref, proj_ref, gidx_ref):
    ap_ref[...] = jnp.exp(logits_ref[...] - m_ref[...]) / s_ref[...]
    proj_ref[...] = bds_ref[...] @ cw_ref[...] + cb_ref[...]
    v = cv_ref[...]
    i = ci_ref[...]
    lane = lax.broadcasted_iota(jnp.int32, (B_, 16), 1)
    acc = jnp.zeros((B_, 16), jnp.int32)
    for t in range(K_):
        m = jnp.max(v, axis=1, keepdims=True)
        wi = jnp.min(jnp.where(v == m, i, _BIGI_), axis=1, keepdims=True)
        acc = jnp.where(lane == t, wi, acc)
        v = jnp.where(i == wi, _NEG_, v)
    gidx_ref[...] = acc


def _finalize_call(logits, m, s, bds, cand_v, cand_i, P):
    return pl.pallas_call(
        _finalize_body,
        out_shape=[
            jax.ShapeDtypeStruct((B_, N_), jnp.float32),
            jax.ShapeDtypeStruct((B_, D_), jnp.float32),
            jax.ShapeDtypeStruct((B_, 16), jnp.int32),
        ],
    )(logits, m, s, bds, P["clip"]["W"], P["clip"]["b"].reshape(1, D_),
      cand_v, cand_i)


def _mixer_body(*refs):
    tk_ref = refs[0]
    mx = refs[1:1 + 8 * MIXL_]
    aaw, aab, abw, abb, acw, acb = refs[1 + 8 * MIXL_:1 + 8 * MIXL_ + 6]
    bsel_ref, psum_ref, af_ref = refs[-3:]

    u = tk_ref[0]
    for l in range(MIXL_):
        t1w, t1b, t2w, t2b, c1w, c1b, c2w, c2b = mx[8 * l:8 * l + 8]
        t = _gelu(u @ t1w[...] + t1b[...]) @ t2w[...] + t2b[...]
        u = u + t
        h = _gelu(lax.dot_general(u, c1w[...], (((0,), (0,)), ((), ())),
                                  preferred_element_type=jnp.float32)
                  + c1b[...])
        ct = lax.dot_general(c2w[...], h, (((0,), (1,)), ((), ())),
                             preferred_element_type=jnp.float32)
        u = u + ct + c2b[...].reshape(K_, 1)

    a2 = jnp.tanh(lax.dot_general(aaw[...], u, (((0,), (0,)), ((), ())),
                                  preferred_element_type=jnp.float32)
                  + aab[...].reshape(DD_, 1))
    s2 = jax.nn.sigmoid(lax.dot_general(abw[...], u, (((0,), (0,)), ((), ())),
                                        preferred_element_type=jnp.float32)
                        + abb[...].reshape(DD_, 1))
    af = jnp.sum(a2 * s2 * acw[...].reshape(DD_, 1), axis=0, keepdims=True)
    af = jax.nn.sigmoid(af + acb[...])
    psum = jnp.sum(tk_ref[0], axis=0, keepdims=True)
    af_ref[0] = af
    psum_ref[0] = psum
    bsel_ref[0] = psum * af


def _mixer_call(topk_feats, P):
    args = [topk_feats]
    for lp in P["mixer"]:
        args += [lp["t1"]["W"], lp["t1"]["b"].reshape(1, DD_),
                 lp["t2"]["W"], lp["t2"]["b"].reshape(1, D_),
                 lp["c1"]["W"], lp["c1"]["b"].reshape(1, DD_),
                 lp["c2"]["W"], lp["c2"]["b"].reshape(1, K_)]
    ga = P["aux_ga"]
    args += [ga["a"]["W"], ga["a"]["b"],
             ga["b"]["W"], ga["b"]["b"],
             ga["c"]["W"].reshape(1, DD_), ga["c"]["b"].reshape(1, 1)]

    in_specs = [pl.BlockSpec((1, K_, D_), lambda b: (b, 0, 0))]
    for a in args[1:]:
        s = a.shape
        in_specs.append(pl.BlockSpec(s, lambda b, s=s: (0,) * len(s)))
    row = pl.BlockSpec((1, 1, D_), lambda b: (b, 0, 0))
    out = pl.pallas_call(
        _mixer_body,
        grid=(B_,),
        in_specs=in_specs,
        out_specs=[row, row, row],
        out_shape=[jax.ShapeDtypeStruct((B_, 1, D_), jnp.float32)] * 3,
    )(*args)
    return tuple(o.reshape(B_, D_) for o in out)


RNA_RB_ = 512
RNA_NRB_ = (NRNA_ + RNA_RB_ - 1) // RNA_RB_


def _rna_layer_body(x_ref, w_ref, b_ref, g_ref, be_ref, o_ref, acc):
    rb = pl.program_id(0)

    @pl.when(rb == 0)
    def _init():
        acc[...] = jnp.zeros_like(acc)

    bound = NRNA_ - rb * RNA_RB_
    xb = x_ref[...]
    wb = w_ref[...]
    lane = lax.broadcasted_iota(jnp.int32, (1, RNA_RB_), 1)
    sub = lax.broadcasted_iota(jnp.int32, (RNA_RB_, 1), 0)
    xb = jnp.where(lane < bound, xb, 0.0)
    wb = jnp.where(sub < bound, wb, 0.0)
    acc[...] += lax.dot_general(xb, wb, (((1,), (0,)), ((), ())),
                                preferred_element_type=jnp.float32)

    @pl.when(rb == RNA_NRB_ - 1)
    def _fin():
        y = acc[...] + b_ref[...]
        o_ref[...] = jnp.maximum(_ln(y, g_ref[...], be_ref[...]), 0.0)


def _rna_layer_call(x, W, b, g, beta):
    return pl.pallas_call(
        _rna_layer_body,
        grid=(RNA_NRB_,),
        in_specs=[
            pl.BlockSpec((B_, RNA_RB_), lambda rb: (0, rb)),
            pl.BlockSpec((RNA_RB_, NRNA_), lambda rb: (rb, 0)),
            pl.BlockSpec((1, NRNA_), lambda rb: (0, 0)),
            pl.BlockSpec((1, NRNA_), lambda rb: (0, 0)),
            pl.BlockSpec((1, NRNA_), lambda rb: (0, 0)),
        ],
        out_specs=pl.BlockSpec((B_, NRNA_), lambda rb: (0, 0)),
        out_shape=jax.ShapeDtypeStruct((B_, NRNA_), jnp.float32),
        scratch_shapes=[pltpu.VMEM((B_, NRNA_), jnp.float32)],
    )(x, W, b.reshape(1, NRNA_), g.reshape(1, NRNA_), beta.reshape(1, NRNA_))


def _rna_out_body(r_ref, w_ref, b_ref, cw_ref, cb_ref, emb_ref, proj_ref):
    emb = r_ref[...] @ w_ref[...] + b_ref[...]
    emb_ref[...] = emb
    proj_ref[...] = emb @ cw_ref[...] + cb_ref[...]


def _rna_out_call(r, P):
    return pl.pallas_call(
        _rna_out_body,
        out_shape=[
            jax.ShapeDtypeStruct((B_, DD_), jnp.float32),
            jax.ShapeDtypeStruct((B_, D_), jnp.float32),
        ],
    )(r, P["rna_out"]["W"], P["rna_out"]["b"].reshape(1, DD_),
      P["clip_rna"]["W"], P["clip_rna"]["b"].reshape(1, D_))


_PART_ = N_ // 8
_NVR_ = _PART_ // 16
_NEG_ = -3.0e38
_BIGI_ = 2**31 - 1


def _sc_mesh():
    return plsc.VectorSubcoreMesh(core_axis_name="c", subcore_axis_name="s")


_SC_CP_ = pltpu.CompilerParams(needs_layout_passes=False)


def _sc_local_topk_body(logits_hbm, outv_hbm, outi_hbm, loc_v, topv_v, topi_v):
    c = lax.axis_index("c")
    s = lax.axis_index("s")
    b = c * 2 + s // 8
    part = s % 8
    base = part * _PART_
    pltpu.sync_copy(logits_hbm.at[b, pl.ds(base, _PART_)], loc_v)

    iota = lax.iota(jnp.int32, 16)
    negs = jnp.full((16,), _NEG_, jnp.float32)
    bigs = jnp.full((16,), _BIGI_, jnp.int32)
    gbase = b * N_ + base

    def pass_body(t, carry):
        topv, topi = carry

        def scan_body(j, c2):
            bv, bi = c2
            v = loc_v[pl.ds(pl.multiple_of(j * 16, 16), 16)]
            i = gbase + j * 16 + iota
            take = v > bv
            return jnp.where(take, v, bv), jnp.where(take, i, bi)

        bv, bi = lax.fori_loop(0, _NVR_, scan_body, (negs, bigs))
        m = jnp.max(bv)
        wi = jnp.min(jnp.where(bv == m, bi, bigs))
        p = wi - gbase
        j0 = pl.multiple_of(jnp.bitwise_and(p, jnp.int32(~15)), 16)
        lane = jnp.bitwise_and(p, jnp.int32(15))
        v = loc_v[pl.ds(j0, 16)]
        loc_v[pl.ds(j0, 16)] = jnp.where(iota == lane, _NEG_, v)
        topv = jnp.where(iota == t, m, topv)
        topi = jnp.where(iota == t, wi, topi)
        return topv, topi

    topv, topi = lax.fori_loop(0, K_, pass_body,
                               (negs, jnp.zeros((16,), jnp.int32)))
    topv_v[...] = topv
    topi_v[...] = topi
    row = b * 8 + part
    pltpu.sync_copy(topv_v, outv_hbm.at[row])
    pltpu.sync_copy(topi_v, outi_hbm.at[row])


def _sc_local_topk(logits):
    call = pl.kernel(
        _sc_local_topk_body,
        out_type=[jax.ShapeDtypeStruct((B_ * 8, 16), jnp.float32),
                  jax.ShapeDtypeStruct((B_ * 8, 16), jnp.int32)],
        mesh=_sc_mesh(),
        compiler_params=_SC_CP_,
        scratch_types=[
            pltpu.VMEM((_PART_,), jnp.float32),
            pltpu.VMEM((16,), jnp.float32),
            pltpu.VMEM((16,), jnp.int32),
        ],
    )
    cv, ci = call(logits)
    return cv.reshape(B_, 128), ci.reshape(B_, 128)


def _sc_gather_body(idx_hbm, feats_hbm, out_hbm, gidx_v, rows_v, sem):
    c = lax.axis_index("c")
    s = lax.axis_index("s")
    b = c * 2 + s // 8
    part = s % 8

    @pl.when(part == 0)
    def _g():
        pltpu.sync_copy(idx_hbm.at[b], gidx_v)
        pltpu.async_copy(feats_hbm.at[gidx_v], rows_v, sem).wait()
        pltpu.sync_copy(rows_v, out_hbm.at[b])


def _sc_gather(gidx, feats):
    call = pl.kernel(
        _sc_gather_body,
        out_type=jax.ShapeDtypeStruct((B_, 16, D_), jnp.float32),
        mesh=_sc_mesh(),
        compiler_params=_SC_CP_,
        scratch_types=[
            pltpu.VMEM((16,), jnp.int32),
            pltpu.VMEM((16, D_), jnp.float32),
            pltpu.SemaphoreType.DMA,
        ],
    )
    return call(gidx, feats.reshape(B_ * N_, D_))[:, :K_, :]


def kernel(feats, feats_deep, rna_seq, params):
    P = params
    logits, m, s, bds = _attn_call(feats_deep, P)
    cand_v, cand_i = _sc_local_topk(logits)
    a_patch, bdp, gidx = _finalize_call(logits, m, s, bds, cand_v, cand_i, P)
    topk_feats = _sc_gather(gidx, feats)
    bsel, psum, a_feat = _mixer_call(topk_feats, P)
    r = _rna_layer_call(rna_seq, P["rna1"]["W"], P["rna1"]["b"],
                        P["rnaln1"]["g"], P["rnaln1"]["beta"])
    r = _rna_layer_call(r, P["rna2"]["W"], P["rna2"]["b"],
                        P["rnaln2"]["g"], P["rnaln2"]["beta"])
    rna_emb, rna_proj = _rna_out_call(r, P)
    return (bsel, psum, bdp, bds, rna_proj, rna_emb, a_feat,
            a_patch.reshape(B_, N_, 1))


# speedup vs baseline: 2.5544x; 1.1451x over previous
import functools

import jax
import jax.numpy as jnp
from jax import lax
from jax.experimental import pallas as pl
from jax.experimental.pallas import tpu as pltpu
from jax.experimental.pallas import tpu_sc as plsc

B_ = 4
N_ = 8192
D_ = 512
DD_ = 256
K_ = 10
NRNA_ = 4848
MIXL_ = 4

NB_ = 8
NBLK_ = N_ // NB_


def _ln(x, g, beta):
    m = x.mean(-1, keepdims=True)
    msq = (x * x).mean(-1, keepdims=True)
    v = jnp.maximum(msq - m * m, 0.0)
    return (x - m) * lax.rsqrt(v + 1e-5) * g + beta


def _gelu(x):
    return x * 0.5 * (1.0 + lax.erf(x * 0.7071067811865476))


def _attn_body(fd_ref, p1w, p1b, g1, be1, p2w, p2b, g2, be2,
               aw, ab, bw, bb, cw, cb,
               logits_ref, m_ref, s_ref, bds_ref,
               m_sc, s_sc, w_sc):
    nb = pl.program_id(1)

    @pl.when(nb == 0)
    def _init():
        m_sc[0, 0] = -1e30
        s_sc[0, 0] = 0.0
        w_sc[...] = jnp.zeros_like(w_sc)

    fd = fd_ref[0]
    x = _gelu(_ln(fd @ p1w[...] + p1b[...], g1[...], be1[...]))
    v = _gelu(_ln(x @ p2w[...] + p2b[...], g2[...], be2[...]))
    a = jnp.tanh(v @ aw[...] + ab[...])
    sgt = jax.nn.sigmoid(v @ bw[...] + bb[...])
    gate = a * sgt
    logit = jnp.sum(gate * cw[...], axis=1, keepdims=True) + cb[...]
    logits_ref[0, 0] = logit

    m_old = m_sc[0, 0]
    mb = jnp.max(logit)
    m_new = jnp.maximum(m_old, mb)
    alpha = jnp.exp(m_old - m_new)
    e = jnp.exp(logit - m_new)
    s_new = s_sc[0, 0] * alpha + jnp.sum(e)
    wv = lax.dot_general(e, v, (((0,), (0,)), ((), ())),
                         preferred_element_type=jnp.float32)
    w_sc[...] = w_sc[...] * alpha + wv
    m_sc[0, 0] = m_new
    s_sc[0, 0] = s_new

    @pl.when(nb == NB_ - 1)
    def _fin():
        m_ref[...] = jnp.reshape(m_new, (1, 1, 1))
        s_ref[...] = jnp.reshape(s_new, (1, 1, 1))
        bds_ref[0] = w_sc[...] / s_new


def _attn_call(feats_deep, P):
    pre1, pre2 = P["pre1"], P["pre2"]
    ga = P["attn_deep"]
    args = (feats_deep,
            pre1["W"], pre1["b"].reshape(1, DD_),
            P["preln1"]["g"].reshape(1, DD_), P["preln1"]["beta"].reshape(1, DD_),
            pre2["W"], pre2["b"].reshape(1, DD_),
            P["preln2"]["g"].reshape(1, DD_), P["preln2"]["beta"].reshape(1, DD_),
            ga["a"]["W"], ga["a"]["b"].reshape(1, DD_),
            ga["b"]["W"], ga["b"]["b"].reshape(1, DD_),
            ga["c"]["W"].reshape(1, DD_), ga["c"]["b"].reshape(1, 1))
    full = lambda s: pl.BlockSpec(s, lambda b, nb: (0,) * len(s))
    out = pl.pallas_call(
        _attn_body,
        grid=(B_, NB_),
        in_specs=[
            pl.BlockSpec((1, NBLK_, DD_), lambda b, nb: (b, nb, 0)),
            full((DD_, DD_)), full((1, DD_)), full((1, DD_)), full((1, DD_)),
            full((DD_, DD_)), full((1, DD_)), full((1, DD_)), full((1, DD_)),
            full((DD_, DD_)), full((1, DD_)),
            full((DD_, DD_)), full((1, DD_)),
            full((1, DD_)), full((1, 1)),
        ],
        out_specs=[
            pl.BlockSpec((1, 1, NBLK_, 1), lambda b, nb: (b, nb, 0, 0)),
            pl.BlockSpec((1, 1, 1), lambda b, nb: (b, 0, 0)),
            pl.BlockSpec((1, 1, 1), lambda b, nb: (b, 0, 0)),
            pl.BlockSpec((1, 1, DD_), lambda b, nb: (b, 0, 0)),
        ],
        out_shape=[
            jax.ShapeDtypeStruct((B_, NB_, NBLK_, 1), jnp.float32),
            jax.ShapeDtypeStruct((B_, 1, 1), jnp.float32),
            jax.ShapeDtypeStruct((B_, 1, 1), jnp.float32),
            jax.ShapeDtypeStruct((B_, 1, DD_), jnp.float32),
        ],
        scratch_shapes=[
            pltpu.SMEM((1, 1), jnp.float32),
            pltpu.SMEM((1, 1), jnp.float32),
            pltpu.VMEM((1, DD_), jnp.float32),
        ],
    )(*args)
    logits, m, s, bds = out
    return (logits.reshape(B_, N_), m.reshape(B_, 1), s.reshape(B_, 1),
            bds.reshape(B_, DD_))


def _mixer_body(*refs):
    tk_ref = refs[0]
    bds_ref, cw_ref, cb_ref = refs[1:4]
    mx = refs[4:4 + 8 * MIXL_]
    aaw, aab, abw, abb, acw, acb = refs[4 + 8 * MIXL_:4 + 8 * MIXL_ + 6]
    bsel_ref, psum_ref, af_ref, proj_ref = refs[-4:]
    proj_ref[0] = bds_ref[0] @ cw_ref[...] + cb_ref[...]

    u = tk_ref[0]
    for l in range(MIXL_):
        t1w, t1b, t2w, t2b, c1w, c1b, c2w, c2b = mx[8 * l:8 * l + 8]
        t = _gelu(u @ t1w[...] + t1b[...]) @ t2w[...] + t2b[...]
        u = u + t
        h = _gelu(lax.dot_general(u, c1w[...], (((0,), (0,)), ((), ())),
                                  preferred_element_type=jnp.float32)
                  + c1b[...])
        ct = lax.dot_general(c2w[...], h, (((0,), (1,)), ((), ())),
                             preferred_element_type=jnp.float32)
        u = u + ct + c2b[...].reshape(K_, 1)

    a2 = jnp.tanh(lax.dot_general(aaw[...], u, (((0,), (0,)), ((), ())),
                                  preferred_element_type=jnp.float32)
                  + aab[...].reshape(DD_, 1))
    s2 = jax.nn.sigmoid(lax.dot_general(abw[...], u, (((0,), (0,)), ((), ())),
                                        preferred_element_type=jnp.float32)
                        + abb[...].reshape(DD_, 1))
    af = jnp.sum(a2 * s2 * acw[...].reshape(DD_, 1), axis=0, keepdims=True)
    af = jax.nn.sigmoid(af + acb[...])
    psum = jnp.sum(tk_ref[0], axis=0, keepdims=True)
    af_ref[0] = af
    psum_ref[0] = psum
    bsel_ref[0] = psum * af


def _mixer_call(topk_feats, bds, P):
    args = [topk_feats, bds.reshape(B_, 1, DD_),
            P["clip"]["W"], P["clip"]["b"].reshape(1, D_)]
    for lp in P["mixer"]:
        args += [lp["t1"]["W"], lp["t1"]["b"].reshape(1, DD_),
                 lp["t2"]["W"], lp["t2"]["b"].reshape(1, D_),
                 lp["c1"]["W"], lp["c1"]["b"].reshape(1, DD_),
                 lp["c2"]["W"], lp["c2"]["b"].reshape(1, K_)]
    ga = P["aux_ga"]
    args += [ga["a"]["W"], ga["a"]["b"],
             ga["b"]["W"], ga["b"]["b"],
             ga["c"]["W"].reshape(1, DD_), ga["c"]["b"].reshape(1, 1)]

    in_specs = [pl.BlockSpec((1, K_, D_), lambda b: (b, 0, 0)),
                pl.BlockSpec((1, 1, DD_), lambda b: (b, 0, 0))]
    for a in args[2:]:
        s = a.shape
        in_specs.append(pl.BlockSpec(s, lambda b, s=s: (0,) * len(s)))
    row = pl.BlockSpec((1, 1, D_), lambda b: (b, 0, 0))
    out = pl.pallas_call(
        _mixer_body,
        grid=(B_,),
        in_specs=in_specs,
        out_specs=[row, row, row, row],
        out_shape=[jax.ShapeDtypeStruct((B_, 1, D_), jnp.float32)] * 4,
    )(*args)
    return tuple(o.reshape(B_, D_) for o in out)


RNA_RB_ = 512
RNA_NRB_ = (NRNA_ + RNA_RB_ - 1) // RNA_RB_


def _rna_layer_body(x_ref, w_ref, b_ref, g_ref, be_ref, o_ref, acc):
    rb = pl.program_id(0)

    @pl.when(rb == 0)
    def _init():
        acc[...] = jnp.zeros_like(acc)

    @pl.when(rb < RNA_NRB_ - 1)
    def _full():
        acc[...] += lax.dot_general(x_ref[...], w_ref[...],
                                    (((1,), (0,)), ((), ())),
                                    preferred_element_type=jnp.float32)

    @pl.when(rb == RNA_NRB_ - 1)
    def _edge():
        bound = NRNA_ - rb * RNA_RB_
        lane = lax.broadcasted_iota(jnp.int32, (1, RNA_RB_), 1)
        sub = lax.broadcasted_iota(jnp.int32, (RNA_RB_, 1), 0)
        xb = jnp.where(lane < bound, x_ref[...], 0.0)
        wb = jnp.where(sub < bound, w_ref[...], 0.0)
        acc[...] += lax.dot_general(xb, wb, (((1,), (0,)), ((), ())),
                                    preferred_element_type=jnp.float32)

    @pl.when(rb == RNA_NRB_ - 1)
    def _fin():
        y = acc[...] + b_ref[...]
        o_ref[...] = jnp.maximum(_ln(y, g_ref[...], be_ref[...]), 0.0)


def _rna_layer_call(x, W, b, g, beta):
    return pl.pallas_call(
        _rna_layer_body,
        grid=(RNA_NRB_,),
        in_specs=[
            pl.BlockSpec((B_, RNA_RB_), lambda rb: (0, rb)),
            pl.BlockSpec((RNA_RB_, NRNA_), lambda rb: (rb, 0)),
            pl.BlockSpec((1, NRNA_), lambda rb: (0, 0)),
            pl.BlockSpec((1, NRNA_), lambda rb: (0, 0)),
            pl.BlockSpec((1, NRNA_), lambda rb: (0, 0)),
        ],
        out_specs=pl.BlockSpec((B_, NRNA_), lambda rb: (0, 0)),
        out_shape=jax.ShapeDtypeStruct((B_, NRNA_), jnp.float32),
        scratch_shapes=[pltpu.VMEM((B_, NRNA_), jnp.float32)],
    )(x, W, b.reshape(1, NRNA_), g.reshape(1, NRNA_), beta.reshape(1, NRNA_))


def _rna_out_body(r_ref, w_ref, b_ref, cw_ref, cb_ref, emb_ref, proj_ref):
    emb = r_ref[...] @ w_ref[...] + b_ref[...]
    emb_ref[...] = emb
    proj_ref[...] = emb @ cw_ref[...] + cb_ref[...]


def _rna_out_call(r, P):
    return pl.pallas_call(
        _rna_out_body,
        out_shape=[
            jax.ShapeDtypeStruct((B_, DD_), jnp.float32),
            jax.ShapeDtypeStruct((B_, D_), jnp.float32),
        ],
    )(r, P["rna_out"]["W"], P["rna_out"]["b"].reshape(1, DD_),
      P["clip_rna"]["W"], P["clip_rna"]["b"].reshape(1, D_))


_PART_ = N_ // 8
_NVR_ = _PART_ // 16
_NEG_ = -3.0e38
_BIGI_ = 2**31 - 1


def _sc_mesh():
    return plsc.VectorSubcoreMesh(core_axis_name="c", subcore_axis_name="s")


_SC_CP_ = pltpu.CompilerParams(needs_layout_passes=False)


def _sc_local_topk_body(logits_hbm, m_hbm, s_hbm, outv_hbm, outi_hbm, ap_hbm,
                        loc_v, ap_v, ms_v, topv_v, topi_v):
    c = lax.axis_index("c")
    s = lax.axis_index("s")
    b = c * 2 + s // 8
    part = s % 8
    base = part * _PART_
    pltpu.sync_copy(logits_hbm.at[b, pl.ds(base, _PART_)], loc_v)

    pltpu.sync_copy(m_hbm.at[b], ms_v)
    mvec = ms_v[...]
    pltpu.sync_copy(s_hbm.at[b], ms_v)
    rinv = 1.0 / ms_v[...]

    def ap_body(j, u):
        off = pl.multiple_of(j * 16, 16)
        ap_v[pl.ds(off, 16)] = jnp.exp(loc_v[pl.ds(off, 16)] - mvec) * rinv
        return u

    lax.fori_loop(0, _NVR_, ap_body, 0)
    pltpu.sync_copy(ap_v, ap_hbm.at[b, pl.ds(base, _PART_)])

    iota = lax.iota(jnp.int32, 16)
    negs = jnp.full((16,), _NEG_, jnp.float32)
    bigs = jnp.full((16,), _BIGI_, jnp.int32)
    gbase = b * N_ + base

    def pass_body(t, carry):
        topv, topi = carry

        def scan_body(j, c2):
            bv, bi = c2
            v = loc_v[pl.ds(pl.multiple_of(j * 16, 16), 16)]
            i = gbase + j * 16 + iota
            take = v > bv
            return jnp.where(take, v, bv), jnp.where(take, i, bi)

        bv, bi = lax.fori_loop(0, _NVR_, scan_body, (negs, bigs))
        m = jnp.max(bv)
        wi = jnp.min(jnp.where(bv == m, bi, bigs))
        p = wi - gbase
        j0 = pl.multiple_of(jnp.bitwise_and(p, jnp.int32(~15)), 16)
        lane = jnp.bitwise_and(p, jnp.int32(15))
        v = loc_v[pl.ds(j0, 16)]
        loc_v[pl.ds(j0, 16)] = jnp.where(iota == lane, _NEG_, v)
        topv = jnp.where(iota == t, m, topv)
        topi = jnp.where(iota == t, wi, topi)
        return topv, topi

    topv, topi = lax.fori_loop(0, K_, pass_body,
                               (negs, jnp.zeros((16,), jnp.int32)))
    topv_v[...] = topv
    topi_v[...] = topi
    row = b * 8 + part
    pltpu.sync_copy(topv_v, outv_hbm.at[row])
    pltpu.sync_copy(topi_v, outi_hbm.at[row])


def _sc_local_topk(logits, m16, s16):
    call = pl.kernel(
        _sc_local_topk_body,
        out_type=[jax.ShapeDtypeStruct((B_ * 8, 16), jnp.float32),
                  jax.ShapeDtypeStruct((B_ * 8, 16), jnp.int32),
                  jax.ShapeDtypeStruct((B_, N_), jnp.float32)],
        mesh=_sc_mesh(),
        compiler_params=_SC_CP_,
        scratch_types=[
            pltpu.VMEM((_PART_,), jnp.float32),
            pltpu.VMEM((_PART_,), jnp.float32),
            pltpu.VMEM((16,), jnp.float32),
            pltpu.VMEM((16,), jnp.float32),
            pltpu.VMEM((16,), jnp.int32),
        ],
    )
    return call(logits, m16, s16)


def _sc_merge_gather_body(cv_hbm, ci_hbm, feats_hbm, out_hbm,
                          candv_v, candi_v, gidx_v, rows_v, sem):
    c = lax.axis_index("c")
    s = lax.axis_index("s")
    b = c * 2 + s // 8
    part = s % 8
    iota = lax.iota(jnp.int32, 16)
    negs = jnp.full((16,), _NEG_, jnp.float32)
    bigs = jnp.full((16,), _BIGI_, jnp.int32)

    @pl.when(part == 0)
    def _merge():
        b0 = pl.multiple_of(b * 8, 8)
        pltpu.sync_copy(cv_hbm.at[pl.ds(b0, 8)], candv_v)
        pltpu.sync_copy(ci_hbm.at[pl.ds(b0, 8)], candi_v)

        def mpass(t, gv):
            def mscan(r, c2):
                bv, bi = c2
                v = candv_v[r]
                i = candi_v[r]
                take = (v > bv) | ((v == bv) & (i < bi))
                return jnp.where(take, v, bv), jnp.where(take, i, bi)

            bv, bi = lax.fori_loop(0, 8, mscan, (negs, bigs))
            m = jnp.max(bv)
            wi = jnp.min(jnp.where(bv == m, bi, bigs))

            def mclear(r, u):
                v = candv_v[r]
                candv_v[r] = jnp.where(candi_v[r] == wi, _NEG_, v)
                return u

            lax.fori_loop(0, 8, mclear, 0)
            return jnp.where(iota == t, wi, gv)

        gv = lax.fori_loop(0, K_, mpass, jnp.zeros((16,), jnp.int32))
        gidx_v[...] = gv
        pltpu.async_copy(feats_hbm.at[gidx_v], rows_v, sem).wait()
        pltpu.sync_copy(rows_v, out_hbm.at[b])


def _sc_merge_gather(cand_v, cand_i, feats):
    call = pl.kernel(
        _sc_merge_gather_body,
        out_type=jax.ShapeDtypeStruct((B_, 16, D_), jnp.float32),
        mesh=_sc_mesh(),
        compiler_params=_SC_CP_,
        scratch_types=[
            pltpu.VMEM((8, 16), jnp.float32),
            pltpu.VMEM((8, 16), jnp.int32),
            pltpu.VMEM((16,), jnp.int32),
            pltpu.VMEM((16, D_), jnp.float32),
            pltpu.SemaphoreType.DMA,
        ],
    )
    return call(cand_v, cand_i, feats.reshape(B_ * N_, D_))[:, :K_, :]


def kernel(feats, feats_deep, rna_seq, params):
    P = params
    logits, m, s, bds = _attn_call(feats_deep, P)
    m16 = jnp.broadcast_to(m, (B_, 16))
    s16 = jnp.broadcast_to(s, (B_, 16))
    cand_v, cand_i, a_patch = _sc_local_topk(logits, m16, s16)
    topk_feats = _sc_merge_gather(cand_v, cand_i, feats)
    bsel, psum, a_feat, bdp = _mixer_call(topk_feats, bds, P)
    r = _rna_layer_call(rna_seq, P["rna1"]["W"], P["rna1"]["b"],
                        P["rnaln1"]["g"], P["rnaln1"]["beta"])
    r = _rna_layer_call(r, P["rna2"]["W"], P["rna2"]["b"],
                        P["rnaln2"]["g"], P["rnaln2"]["beta"])
    rna_emb, rna_proj = _rna_out_call(r, P)
    return (bsel, psum, bdp, bds, rna_proj, rna_emb, a_feat,
            a_patch.reshape(B_, N_, 1))


# speedup vs baseline: 2.7303x; 1.0688x over previous
import functools

import jax
import jax.numpy as jnp
from jax import lax
from jax.experimental import pallas as pl
from jax.experimental.pallas import tpu as pltpu
from jax.experimental.pallas import tpu_sc as plsc

B_ = 4
N_ = 8192
D_ = 512
DD_ = 256
K_ = 10
NRNA_ = 4848
MIXL_ = 4

NB_ = 8
NBLK_ = N_ // NB_


def _ln(x, g, beta):
    m = x.mean(-1, keepdims=True)
    msq = (x * x).mean(-1, keepdims=True)
    v = jnp.maximum(msq - m * m, 0.0)
    return (x - m) * lax.rsqrt(v + 1e-5) * g + beta


def _gelu(x):
    return x * 0.5 * (1.0 + lax.erf(x * 0.7071067811865476))


def _fused_body(fd_ref, p1w, p1b, g1, be1, p2w, p2b, g2, be2,
                aw, ab, bw, bb, cw, cb,
                x1_ref, w1_ref, b1_ref, g1r_ref, be1r_ref,
                w2_ref, b2_ref, g2r_ref, be2r_ref,
                wo_ref, bo_ref, cw2_ref, cb2_ref,
                logits_ref, m_ref, s_ref, bds_ref, emb_ref, proj_ref,
                m_sc, s_sc, w_sc, acc1, r1_sc, acc2):
    i = pl.program_id(0)
    nb = i % NB_

    @pl.when(i < RNA_NRB_ - 1)
    def _rna1():
        @pl.when(i == 0)
        def _z():
            acc1[...] = jnp.zeros_like(acc1)

        acc1[...] += lax.dot_general(x1_ref[...], w1_ref[...],
                                     (((1,), (0,)), ((), ())),
                                     preferred_element_type=jnp.float32)

    @pl.when(i == RNA_NRB_ - 1)
    def _rna1_fin():
        bound = NRNA_ - (RNA_NRB_ - 1) * RNA_RB_
        lane = lax.broadcasted_iota(jnp.int32, (1, RNA_RB_), 1)
        sub = lax.broadcasted_iota(jnp.int32, (RNA_RB_, 1), 0)
        xb = jnp.where(lane < bound, x1_ref[...], 0.0)
        wb = jnp.where(sub < bound, w1_ref[...], 0.0)
        y = acc1[...] + lax.dot_general(xb, wb, (((1,), (0,)), ((), ())),
                                        preferred_element_type=jnp.float32)
        y = y + b1_ref[...]
        r1_sc[...] = jnp.maximum(_ln(y, g1r_ref[...], be1r_ref[...]), 0.0)

    @pl.when((i >= RNA_NRB_) & (i < 2 * RNA_NRB_))
    def _rna2():
        k = i - RNA_NRB_
        yb = lax.dot_general(r1_sc[...], w2_ref[...], (((1,), (0,)), ((), ())),
                             preferred_element_type=jnp.float32)
        acc2[:, pl.ds(pl.multiple_of(k * RNA_RB_, 128), RNA_RB_)] = (
            yb + b2_ref[...])

    @pl.when(i == 2 * RNA_NRB_)
    def _rna2_fin():
        lane = lax.broadcasted_iota(jnp.int32, (1, _NPAD_), 1)
        valid = lane < NRNA_
        y = jnp.where(valid, acc2[...], 0.0)
        mu = jnp.sum(y, axis=1, keepdims=True) / NRNA_
        msq = jnp.sum(y * y, axis=1, keepdims=True) / NRNA_
        var = jnp.maximum(msq - mu * mu, 0.0)
        r2 = (y - mu) * lax.rsqrt(var + 1e-5) * g2r_ref[...] + be2r_ref[...]
        r2 = jnp.where(valid, jnp.maximum(r2, 0.0), 0.0)
        emb = r2 @ wo_ref[...] + bo_ref[...]
        emb_ref[...] = emb
        proj_ref[...] = emb @ cw2_ref[...] + cb2_ref[...]

    @pl.when(nb == 0)
    def _init():
        m_sc[0, 0] = -1e30
        s_sc[0, 0] = 0.0
        w_sc[...] = jnp.zeros_like(w_sc)

    fd = fd_ref[0]
    x = _gelu(_ln(fd @ p1w[...] + p1b[...], g1[...], be1[...]))
    v = _gelu(_ln(x @ p2w[...] + p2b[...], g2[...], be2[...]))
    a = jnp.tanh(v @ aw[...] + ab[...])
    sgt = jax.nn.sigmoid(v @ bw[...] + bb[...])
    gate = a * sgt
    logit = jnp.sum(gate * cw[...], axis=1, keepdims=True) + cb[...]
    logits_ref[0, 0] = logit

    m_old = m_sc[0, 0]
    mb = jnp.max(logit)
    m_new = jnp.maximum(m_old, mb)
    alpha = jnp.exp(m_old - m_new)
    e = jnp.exp(logit - m_new)
    s_new = s_sc[0, 0] * alpha + jnp.sum(e)
    wv = lax.dot_general(e, v, (((0,), (0,)), ((), ())),
                         preferred_element_type=jnp.float32)
    w_sc[...] = w_sc[...] * alpha + wv
    m_sc[0, 0] = m_new
    s_sc[0, 0] = s_new

    @pl.when(nb == NB_ - 1)
    def _fin():
        m_ref[...] = jnp.reshape(m_new, (1, 1, 1))
        s_ref[...] = jnp.reshape(s_new, (1, 1, 1))
        bds_ref[0] = w_sc[...] / s_new


def _fused_call(feats_deep, rna_seq, P):
    pre1, pre2 = P["pre1"], P["pre2"]
    ga = P["attn_deep"]
    wo_p = jnp.zeros((_NPAD_, DD_), jnp.float32).at[:NRNA_].set(
        P["rna_out"]["W"])
    g2_p = jnp.zeros((1, _NPAD_), jnp.float32).at[:, :NRNA_].set(
        P["rnaln2"]["g"].reshape(1, NRNA_))
    be2_p = jnp.zeros((1, _NPAD_), jnp.float32).at[:, :NRNA_].set(
        P["rnaln2"]["beta"].reshape(1, NRNA_))
    args = (feats_deep,
            pre1["W"], pre1["b"].reshape(1, DD_),
            P["preln1"]["g"].reshape(1, DD_), P["preln1"]["beta"].reshape(1, DD_),
            pre2["W"], pre2["b"].reshape(1, DD_),
            P["preln2"]["g"].reshape(1, DD_), P["preln2"]["beta"].reshape(1, DD_),
            ga["a"]["W"], ga["a"]["b"].reshape(1, DD_),
            ga["b"]["W"], ga["b"]["b"].reshape(1, DD_),
            ga["c"]["W"].reshape(1, DD_), ga["c"]["b"].reshape(1, 1),
            rna_seq, P["rna1"]["W"], P["rna1"]["b"].reshape(1, NRNA_),
            P["rnaln1"]["g"].reshape(1, NRNA_),
            P["rnaln1"]["beta"].reshape(1, NRNA_),
            P["rna2"]["W"], P["rna2"]["b"].reshape(1, NRNA_),
            g2_p, be2_p,
            wo_p, P["rna_out"]["b"].reshape(1, DD_),
            P["clip_rna"]["W"], P["clip_rna"]["b"].reshape(1, D_))
    full = lambda s: pl.BlockSpec(s, lambda i, s=s: (0,) * len(s))
    out = pl.pallas_call(
        _fused_body,
        grid=(B_ * NB_,),
        in_specs=[
            pl.BlockSpec((1, NBLK_, DD_), lambda i: (i // NB_, i % NB_, 0)),
            full((DD_, DD_)), full((1, DD_)), full((1, DD_)), full((1, DD_)),
            full((DD_, DD_)), full((1, DD_)), full((1, DD_)), full((1, DD_)),
            full((DD_, DD_)), full((1, DD_)),
            full((DD_, DD_)), full((1, DD_)),
            full((1, DD_)), full((1, 1)),
            pl.BlockSpec((B_, RNA_RB_),
                         lambda i: (0, jnp.minimum(i, RNA_NRB_ - 1))),
            pl.BlockSpec((RNA_RB_, NRNA_),
                         lambda i: (jnp.minimum(i, RNA_NRB_ - 1), 0)),
            full((1, NRNA_)), full((1, NRNA_)), full((1, NRNA_)),
            pl.BlockSpec((NRNA_, RNA_RB_),
                         lambda i: (0, jnp.clip(i - RNA_NRB_, 0,
                                                RNA_NRB_ - 1))),
            pl.BlockSpec((1, RNA_RB_),
                         lambda i: (0, jnp.clip(i - RNA_NRB_, 0,
                                                RNA_NRB_ - 1))),
            full((1, _NPAD_)), full((1, _NPAD_)),
            full((_NPAD_, DD_)), full((1, DD_)),
            full((DD_, D_)), full((1, D_)),
        ],
        out_specs=[
            pl.BlockSpec((1, 1, NBLK_, 1), lambda i: (i // NB_, i % NB_, 0, 0)),
            pl.BlockSpec((1, 1, 1), lambda i: (i // NB_, 0, 0)),
            pl.BlockSpec((1, 1, 1), lambda i: (i // NB_, 0, 0)),
            pl.BlockSpec((1, 1, DD_), lambda i: (i // NB_, 0, 0)),
            full((B_, DD_)),
            full((B_, D_)),
        ],
        out_shape=[
            jax.ShapeDtypeStruct((B_, NB_, NBLK_, 1), jnp.float32),
            jax.ShapeDtypeStruct((B_, 1, 1), jnp.float32),
            jax.ShapeDtypeStruct((B_, 1, 1), jnp.float32),
            jax.ShapeDtypeStruct((B_, 1, DD_), jnp.float32),
            jax.ShapeDtypeStruct((B_, DD_), jnp.float32),
            jax.ShapeDtypeStruct((B_, D_), jnp.float32),
        ],
        scratch_shapes=[
            pltpu.SMEM((1, 1), jnp.float32),
            pltpu.SMEM((1, 1), jnp.float32),
            pltpu.VMEM((1, DD_), jnp.float32),
            pltpu.VMEM((B_, NRNA_), jnp.float32),
            pltpu.VMEM((B_, NRNA_), jnp.float32),
            pltpu.VMEM((B_, _NPAD_), jnp.float32),
        ],
    )(*args)
    logits, m, s, bds, emb, proj = out
    return (logits.reshape(B_, N_), m.reshape(B_, 1), s.reshape(B_, 1),
            bds.reshape(B_, DD_), emb, proj)


def _mixer_body(*refs):
    tk_ref = refs[0]
    bds_ref, cw_ref, cb_ref = refs[1:4]
    mx = refs[4:4 + 8 * MIXL_]
    aaw, aab, abw, abb, acw, acb = refs[4 + 8 * MIXL_:4 + 8 * MIXL_ + 6]
    bsel_ref, psum_ref, af_ref, proj_ref = refs[-4:]
    proj_ref[0] = bds_ref[0] @ cw_ref[...] + cb_ref[...]

    u = tk_ref[0]
    for l in range(MIXL_):
        t1w, t1b, t2w, t2b, c1w, c1b, c2w, c2b = mx[8 * l:8 * l + 8]
        t = _gelu(u @ t1w[...] + t1b[...]) @ t2w[...] + t2b[...]
        u = u + t
        h = _gelu(lax.dot_general(u, c1w[...], (((0,), (0,)), ((), ())),
                                  preferred_element_type=jnp.float32)
                  + c1b[...])
        ct = lax.dot_general(c2w[...], h, (((0,), (1,)), ((), ())),
                             preferred_element_type=jnp.float32)
        u = u + ct + c2b[...].reshape(K_, 1)

    a2 = jnp.tanh(lax.dot_general(aaw[...], u, (((0,), (0,)), ((), ())),
                                  preferred_element_type=jnp.float32)
                  + aab[...].reshape(DD_, 1))
    s2 = jax.nn.sigmoid(lax.dot_general(abw[...], u, (((0,), (0,)), ((), ())),
                                        preferred_element_type=jnp.float32)
                        + abb[...].reshape(DD_, 1))
    af = jnp.sum(a2 * s2 * acw[...].reshape(DD_, 1), axis=0, keepdims=True)
    af = jax.nn.sigmoid(af + acb[...])
    psum = jnp.sum(tk_ref[0], axis=0, keepdims=True)
    af_ref[0] = af
    psum_ref[0] = psum
    bsel_ref[0] = psum * af


def _mixer_call(topk_feats, bds, P):
    args = [topk_feats, bds.reshape(B_, 1, DD_),
            P["clip"]["W"], P["clip"]["b"].reshape(1, D_)]
    for lp in P["mixer"]:
        args += [lp["t1"]["W"], lp["t1"]["b"].reshape(1, DD_),
                 lp["t2"]["W"], lp["t2"]["b"].reshape(1, D_),
                 lp["c1"]["W"], lp["c1"]["b"].reshape(1, DD_),
                 lp["c2"]["W"], lp["c2"]["b"].reshape(1, K_)]
    ga = P["aux_ga"]
    args += [ga["a"]["W"], ga["a"]["b"],
             ga["b"]["W"], ga["b"]["b"],
             ga["c"]["W"].reshape(1, DD_), ga["c"]["b"].reshape(1, 1)]

    in_specs = [pl.BlockSpec((1, K_, D_), lambda b: (b, 0, 0)),
                pl.BlockSpec((1, 1, DD_), lambda b: (b, 0, 0))]
    for a in args[2:]:
        s = a.shape
        in_specs.append(pl.BlockSpec(s, lambda b, s=s: (0,) * len(s)))
    row = pl.BlockSpec((1, 1, D_), lambda b: (b, 0, 0))
    out = pl.pallas_call(
        _mixer_body,
        grid=(B_,),
        in_specs=in_specs,
        out_specs=[row, row, row, row],
        out_shape=[jax.ShapeDtypeStruct((B_, 1, D_), jnp.float32)] * 4,
    )(*args)
    return tuple(o.reshape(B_, D_) for o in out)


RNA_RB_ = 512
RNA_NRB_ = (NRNA_ + RNA_RB_ - 1) // RNA_RB_
_NPAD_ = RNA_NRB_ * RNA_RB_


_PART_ = N_ // 8
_NVR_ = _PART_ // 16
_NEG_ = -3.0e38
_BIGI_ = 2**31 - 1


def _sc_mesh():
    return plsc.VectorSubcoreMesh(core_axis_name="c", subcore_axis_name="s")


_SC_CP_ = pltpu.CompilerParams(needs_layout_passes=False)


def _sc_local_topk_body(logits_hbm, m_hbm, s_hbm, outv_hbm, outi_hbm, ap_hbm,
                        loc_v, ap_v, ms_v, topv_v, topi_v):
    c = lax.axis_index("c")
    s = lax.axis_index("s")
    b = c * 2 + s // 8
    part = s % 8
    base = part * _PART_
    pltpu.sync_copy(logits_hbm.at[b, pl.ds(base, _PART_)], loc_v)

    pltpu.sync_copy(m_hbm.at[b], ms_v)
    mvec = ms_v[...]
    pltpu.sync_copy(s_hbm.at[b], ms_v)
    rinv = 1.0 / ms_v[...]

    def ap_body(j, u):
        off = pl.multiple_of(j * 16, 16)
        ap_v[pl.ds(off, 16)] = jnp.exp(loc_v[pl.ds(off, 16)] - mvec) * rinv
        return u

    lax.fori_loop(0, _NVR_, ap_body, 0)
    pltpu.sync_copy(ap_v, ap_hbm.at[b, pl.ds(base, _PART_)])

    iota = lax.iota(jnp.int32, 16)
    negs = jnp.full((16,), _NEG_, jnp.float32)
    bigs = jnp.full((16,), _BIGI_, jnp.int32)
    gbase = b * N_ + base

    def pass_body(t, carry):
        topv, topi = carry

        def scan_body(j, c2):
            bv, bi = c2
            v = loc_v[pl.ds(pl.multiple_of(j * 16, 16), 16)]
            i = gbase + j * 16 + iota
            take = v > bv
            return jnp.where(take, v, bv), jnp.where(take, i, bi)

        bv, bi = lax.fori_loop(0, _NVR_, scan_body, (negs, bigs))
        m = jnp.max(bv)
        wi = jnp.min(jnp.where(bv == m, bi, bigs))
        p = wi - gbase
        j0 = pl.multiple_of(jnp.bitwise_and(p, jnp.int32(~15)), 16)
        lane = jnp.bitwise_and(p, jnp.int32(15))
        v = loc_v[pl.ds(j0, 16)]
        loc_v[pl.ds(j0, 16)] = jnp.where(iota == lane, _NEG_, v)
        topv = jnp.where(iota == t, m, topv)
        topi = jnp.where(iota == t, wi, topi)
        return topv, topi

    topv, topi = lax.fori_loop(0, K_, pass_body,
                               (negs, jnp.zeros((16,), jnp.int32)))
    topv_v[...] = topv
    topi_v[...] = topi
    row = b * 8 + part
    pltpu.sync_copy(topv_v, outv_hbm.at[row])
    pltpu.sync_copy(topi_v, outi_hbm.at[row])


def _sc_local_topk(logits, m16, s16):
    call = pl.kernel(
        _sc_local_topk_body,
        out_type=[jax.ShapeDtypeStruct((B_ * 8, 16), jnp.float32),
                  jax.ShapeDtypeStruct((B_ * 8, 16), jnp.int32),
                  jax.ShapeDtypeStruct((B_, N_), jnp.float32)],
        mesh=_sc_mesh(),
        compiler_params=_SC_CP_,
        scratch_types=[
            pltpu.VMEM((_PART_,), jnp.float32),
            pltpu.VMEM((_PART_,), jnp.float32),
            pltpu.VMEM((16,), jnp.float32),
            pltpu.VMEM((16,), jnp.float32),
            pltpu.VMEM((16,), jnp.int32),
        ],
    )
    return call(logits, m16, s16)


def _sc_merge_gather_body(cv_hbm, ci_hbm, feats_hbm, out_hbm,
                          candv_v, candi_v, gidx_v, rows_v, sem):
    c = lax.axis_index("c")
    s = lax.axis_index("s")
    b = c * 2 + s // 8
    part = s % 8
    iota = lax.iota(jnp.int32, 16)
    negs = jnp.full((16,), _NEG_, jnp.float32)
    bigs = jnp.full((16,), _BIGI_, jnp.int32)

    @pl.when(part == 0)
    def _merge():
        b0 = pl.multiple_of(b * 8, 8)
        pltpu.sync_copy(cv_hbm.at[pl.ds(b0, 8)], candv_v)
        pltpu.sync_copy(ci_hbm.at[pl.ds(b0, 8)], candi_v)

        def mpass(t, gv):
            def mscan(r, c2):
                bv, bi = c2
                v = candv_v[r]
                i = candi_v[r]
                take = (v > bv) | ((v == bv) & (i < bi))
                return jnp.where(take, v, bv), jnp.where(take, i, bi)

            bv, bi = lax.fori_loop(0, 8, mscan, (negs, bigs))
            m = jnp.max(bv)
            wi = jnp.min(jnp.where(bv == m, bi, bigs))

            def mclear(r, u):
                v = candv_v[r]
                candv_v[r] = jnp.where(candi_v[r] == wi, _NEG_, v)
                return u

            lax.fori_loop(0, 8, mclear, 0)
            return jnp.where(iota == t, wi, gv)

        gv = lax.fori_loop(0, K_, mpass, jnp.zeros((16,), jnp.int32))
        gidx_v[...] = gv
        pltpu.async_copy(feats_hbm.at[gidx_v], rows_v, sem).wait()
        pltpu.sync_copy(rows_v, out_hbm.at[b])


def _sc_merge_gather(cand_v, cand_i, feats):
    call = pl.kernel(
        _sc_merge_gather_body,
        out_type=jax.ShapeDtypeStruct((B_, 16, D_), jnp.float32),
        mesh=_sc_mesh(),
        compiler_params=_SC_CP_,
        scratch_types=[
            pltpu.VMEM((8, 16), jnp.float32),
            pltpu.VMEM((8, 16), jnp.int32),
            pltpu.VMEM((16,), jnp.int32),
            pltpu.VMEM((16, D_), jnp.float32),
            pltpu.SemaphoreType.DMA,
        ],
    )
    return call(cand_v, cand_i, feats.reshape(B_ * N_, D_))[:, :K_, :]


def kernel(feats, feats_deep, rna_seq, params):
    P = params
    logits, m, s, bds, rna_emb, rna_proj = _fused_call(feats_deep, rna_seq, P)
    m16 = jnp.broadcast_to(m, (B_, 16))
    s16 = jnp.broadcast_to(s, (B_, 16))
    cand_v, cand_i, a_patch = _sc_local_topk(logits, m16, s16)
    topk_feats = _sc_merge_gather(cand_v, cand_i, feats)
    bsel, psum, a_feat, bdp = _mixer_call(topk_feats, bds, P)
    return (bsel, psum, bdp, bds, rna_proj, rna_emb, a_feat,
            a_patch.reshape(B_, N_, 1))


# speedup vs baseline: 2.7701x; 1.0146x over previous
import functools

import jax
import jax.numpy as jnp
from jax import lax
from jax.experimental import pallas as pl
from jax.experimental.pallas import tpu as pltpu
from jax.experimental.pallas import tpu_sc as plsc

B_ = 4
N_ = 8192
D_ = 512
DD_ = 256
K_ = 10
NRNA_ = 4848
MIXL_ = 4

NB_ = 8
NBLK_ = N_ // NB_


def _ln(x, g, beta):
    m = x.mean(-1, keepdims=True)
    msq = (x * x).mean(-1, keepdims=True)
    v = jnp.maximum(msq - m * m, 0.0)
    return (x - m) * lax.rsqrt(v + 1e-5) * g + beta


def _gelu(x):
    return x * 0.5 * (1.0 + lax.erf(x * 0.7071067811865476))


def _fused_body(fd_ref, p1w, p1b, g1, be1, p2w, p2b, g2, be2,
                aw, ab, bw, bb, cw, cb,
                x1_ref, w1_ref, b1_ref, g1r_ref, be1r_ref,
                w2_ref, b2_ref, g2r_ref, be2r_ref,
                wo_ref, bo_ref, cw2_ref, cb2_ref,
                logits_ref, m_ref, s_ref, bds_ref, emb_ref, proj_ref,
                m_sc, s_sc, w_sc, acc1, r1_sc, acc2):
    i = pl.program_id(0)
    nb = i % NB_

    @pl.when(i < RNA_NRB_ - 1)
    def _rna1():
        @pl.when(i == 0)
        def _z():
            acc1[...] = jnp.zeros_like(acc1)

        acc1[...] += lax.dot_general(x1_ref[...], w1_ref[...],
                                     (((1,), (0,)), ((), ())),
                                     preferred_element_type=jnp.float32)

    @pl.when(i == RNA_NRB_ - 1)
    def _rna1_fin():
        bound = NRNA_ - (RNA_NRB_ - 1) * RNA_RB_
        lane = lax.broadcasted_iota(jnp.int32, (1, RNA_RB_), 1)
        sub = lax.broadcasted_iota(jnp.int32, (RNA_RB_, 1), 0)
        xb = jnp.where(lane < bound, x1_ref[...], 0.0)
        wb = jnp.where(sub < bound, w1_ref[...], 0.0)
        y = acc1[...] + lax.dot_general(xb, wb, (((1,), (0,)), ((), ())),
                                        preferred_element_type=jnp.float32)
        y = y + b1_ref[...]
        r1_sc[...] = jnp.maximum(_ln(y, g1r_ref[...], be1r_ref[...]), 0.0)

    @pl.when((i >= RNA_NRB_) & (i < 2 * RNA_NRB_))
    def _rna2():
        k = i - RNA_NRB_
        yb = lax.dot_general(r1_sc[...], w2_ref[...], (((1,), (0,)), ((), ())),
                             preferred_element_type=jnp.float32)
        acc2[:, pl.ds(pl.multiple_of(k * RNA_RB_, 128), RNA_RB_)] = (
            yb + b2_ref[...])

    @pl.when(i == 2 * RNA_NRB_)
    def _rna2_fin():
        lane = lax.broadcasted_iota(jnp.int32, (1, _NPAD_), 1)
        valid = lane < NRNA_
        y = jnp.where(valid, acc2[...], 0.0)
        mu = jnp.sum(y, axis=1, keepdims=True) / NRNA_
        msq = jnp.sum(y * y, axis=1, keepdims=True) / NRNA_
        var = jnp.maximum(msq - mu * mu, 0.0)
        r2 = (y - mu) * lax.rsqrt(var + 1e-5) * g2r_ref[...] + be2r_ref[...]
        r2 = jnp.where(valid, jnp.maximum(r2, 0.0), 0.0)
        emb = r2 @ wo_ref[...] + bo_ref[...]
        emb_ref[...] = emb
        proj_ref[...] = emb @ cw2_ref[...] + cb2_ref[...]

    @pl.when(nb == 0)
    def _init():
        m_sc[0, 0] = -1e30
        s_sc[0, 0] = 0.0
        w_sc[...] = jnp.zeros_like(w_sc)

    fd = fd_ref[0]
    x = _gelu(_ln(fd @ p1w[...] + p1b[...], g1[...], be1[...]))
    v = _gelu(_ln(x @ p2w[...] + p2b[...], g2[...], be2[...]))
    a = jnp.tanh(v @ aw[...] + ab[...])
    sgt = jax.nn.sigmoid(v @ bw[...] + bb[...])
    gate = a * sgt
    logit = jnp.sum(gate * cw[...], axis=1, keepdims=True) + cb[...]
    logits_ref[0, 0] = logit

    m_old = m_sc[0, 0]
    mb = jnp.max(logit)
    m_new = jnp.maximum(m_old, mb)
    alpha = jnp.exp(m_old - m_new)
    e = jnp.exp(logit - m_new)
    s_new = s_sc[0, 0] * alpha + jnp.sum(e)
    wv = lax.dot_general(e, v, (((0,), (0,)), ((), ())),
                         preferred_element_type=jnp.float32)
    w_sc[...] = w_sc[...] * alpha + wv
    m_sc[0, 0] = m_new
    s_sc[0, 0] = s_new

    @pl.when(nb == NB_ - 1)
    def _fin():
        m_ref[...] = jnp.reshape(m_new, (1, 1, 1))
        s_ref[...] = jnp.reshape(s_new, (1, 1, 1))
        bds_ref[0] = w_sc[...] / s_new


def _fused_call(feats_deep, rna_seq, P):
    pre1, pre2 = P["pre1"], P["pre2"]
    ga = P["attn_deep"]
    wo_p = jnp.zeros((_NPAD_, DD_), jnp.float32).at[:NRNA_].set(
        P["rna_out"]["W"])
    g2_p = jnp.zeros((1, _NPAD_), jnp.float32).at[:, :NRNA_].set(
        P["rnaln2"]["g"].reshape(1, NRNA_))
    be2_p = jnp.zeros((1, _NPAD_), jnp.float32).at[:, :NRNA_].set(
        P["rnaln2"]["beta"].reshape(1, NRNA_))
    args = (feats_deep,
            pre1["W"], pre1["b"].reshape(1, DD_),
            P["preln1"]["g"].reshape(1, DD_), P["preln1"]["beta"].reshape(1, DD_),
            pre2["W"], pre2["b"].reshape(1, DD_),
            P["preln2"]["g"].reshape(1, DD_), P["preln2"]["beta"].reshape(1, DD_),
            ga["a"]["W"], ga["a"]["b"].reshape(1, DD_),
            ga["b"]["W"], ga["b"]["b"].reshape(1, DD_),
            ga["c"]["W"].reshape(1, DD_), ga["c"]["b"].reshape(1, 1),
            rna_seq, P["rna1"]["W"], P["rna1"]["b"].reshape(1, NRNA_),
            P["rnaln1"]["g"].reshape(1, NRNA_),
            P["rnaln1"]["beta"].reshape(1, NRNA_),
            P["rna2"]["W"], P["rna2"]["b"].reshape(1, NRNA_),
            g2_p, be2_p,
            wo_p, P["rna_out"]["b"].reshape(1, DD_),
            P["clip_rna"]["W"], P["clip_rna"]["b"].reshape(1, D_))
    full = lambda s: pl.BlockSpec(s, lambda i, s=s: (0,) * len(s))
    out = pl.pallas_call(
        _fused_body,
        grid=(B_ * NB_,),
        in_specs=[
            pl.BlockSpec((1, NBLK_, DD_), lambda i: (i // NB_, i % NB_, 0)),
            full((DD_, DD_)), full((1, DD_)), full((1, DD_)), full((1, DD_)),
            full((DD_, DD_)), full((1, DD_)), full((1, DD_)), full((1, DD_)),
            full((DD_, DD_)), full((1, DD_)),
            full((DD_, DD_)), full((1, DD_)),
            full((1, DD_)), full((1, 1)),
            pl.BlockSpec((B_, RNA_RB_),
                         lambda i: (0, jnp.minimum(i, RNA_NRB_ - 1))),
            pl.BlockSpec((RNA_RB_, NRNA_),
                         lambda i: (jnp.minimum(i, RNA_NRB_ - 1), 0)),
            full((1, NRNA_)), full((1, NRNA_)), full((1, NRNA_)),
            pl.BlockSpec((NRNA_, RNA_RB_),
                         lambda i: (0, jnp.clip(i - RNA_NRB_, 0,
                                                RNA_NRB_ - 1))),
            pl.BlockSpec((1, RNA_RB_),
                         lambda i: (0, jnp.clip(i - RNA_NRB_, 0,
                                                RNA_NRB_ - 1))),
            full((1, _NPAD_)), full((1, _NPAD_)),
            full((_NPAD_, DD_)), full((1, DD_)),
            full((DD_, D_)), full((1, D_)),
        ],
        out_specs=[
            pl.BlockSpec((1, 1, NBLK_, 1), lambda i: (i // NB_, i % NB_, 0, 0)),
            pl.BlockSpec((1, 1, 1), lambda i: (i // NB_, 0, 0)),
            pl.BlockSpec((1, 1, 1), lambda i: (i // NB_, 0, 0)),
            pl.BlockSpec((1, 1, DD_), lambda i: (i // NB_, 0, 0)),
            full((B_, DD_)),
            full((B_, D_)),
        ],
        out_shape=[
            jax.ShapeDtypeStruct((B_, NB_, NBLK_, 1), jnp.float32),
            jax.ShapeDtypeStruct((B_, 1, 1), jnp.float32),
            jax.ShapeDtypeStruct((B_, 1, 1), jnp.float32),
            jax.ShapeDtypeStruct((B_, 1, DD_), jnp.float32),
            jax.ShapeDtypeStruct((B_, DD_), jnp.float32),
            jax.ShapeDtypeStruct((B_, D_), jnp.float32),
        ],
        scratch_shapes=[
            pltpu.SMEM((1, 1), jnp.float32),
            pltpu.SMEM((1, 1), jnp.float32),
            pltpu.VMEM((1, DD_), jnp.float32),
            pltpu.VMEM((B_, NRNA_), jnp.float32),
            pltpu.VMEM((B_, NRNA_), jnp.float32),
            pltpu.VMEM((B_, _NPAD_), jnp.float32),
        ],
    )(*args)
    logits, m, s, bds, emb, proj = out
    return (logits.reshape(B_, N_), m.reshape(B_, 1), s.reshape(B_, 1),
            bds.reshape(B_, DD_), emb, proj)


def _mixer_body(*refs):
    tk_ref = refs[0]
    bds_ref, cw_ref, cb_ref = refs[1:4]
    mx = refs[4:4 + 8 * MIXL_]
    aaw, aab, abw, abb, acw, acb = refs[4 + 8 * MIXL_:4 + 8 * MIXL_ + 6]
    bsel_ref, psum_ref, af_ref, proj_ref = refs[-4:]
    proj_ref[0] = bds_ref[0] @ cw_ref[...] + cb_ref[...]

    u = tk_ref[0]
    for l in range(MIXL_):
        t1w, t1b, t2w, t2b, c1w, c1b, c2w, c2b = mx[8 * l:8 * l + 8]
        t = _gelu(u @ t1w[...] + t1b[...]) @ t2w[...] + t2b[...]
        u = u + t
        h = _gelu(lax.dot_general(u, c1w[...], (((0,), (0,)), ((), ())),
                                  preferred_element_type=jnp.float32)
                  + c1b[...])
        ct = lax.dot_general(c2w[...], h, (((0,), (1,)), ((), ())),
                             preferred_element_type=jnp.float32)
        u = u + ct + c2b[...].reshape(K_, 1)

    a2 = jnp.tanh(lax.dot_general(aaw[...], u, (((0,), (0,)), ((), ())),
                                  preferred_element_type=jnp.float32)
                  + aab[...].reshape(DD_, 1))
    s2 = jax.nn.sigmoid(lax.dot_general(abw[...], u, (((0,), (0,)), ((), ())),
                                        preferred_element_type=jnp.float32)
                        + abb[...].reshape(DD_, 1))
    af = jnp.sum(a2 * s2 * acw[...].reshape(DD_, 1), axis=0, keepdims=True)
    af = jax.nn.sigmoid(af + acb[...])
    psum = jnp.sum(tk_ref[0], axis=0, keepdims=True)
    af_ref[0] = af
    psum_ref[0] = psum
    bsel_ref[0] = psum * af


def _mixer_call(topk_feats, bds, P):
    args = [topk_feats, bds.reshape(B_, 1, DD_),
            P["clip"]["W"], P["clip"]["b"].reshape(1, D_)]
    for lp in P["mixer"]:
        args += [lp["t1"]["W"], lp["t1"]["b"].reshape(1, DD_),
                 lp["t2"]["W"], lp["t2"]["b"].reshape(1, D_),
                 lp["c1"]["W"], lp["c1"]["b"].reshape(1, DD_),
                 lp["c2"]["W"], lp["c2"]["b"].reshape(1, K_)]
    ga = P["aux_ga"]
    args += [ga["a"]["W"], ga["a"]["b"],
             ga["b"]["W"], ga["b"]["b"],
             ga["c"]["W"].reshape(1, DD_), ga["c"]["b"].reshape(1, 1)]

    in_specs = [pl.BlockSpec((1, K_, D_), lambda b: (b, 0, 0)),
                pl.BlockSpec((1, 1, DD_), lambda b: (b, 0, 0))]
    for a in args[2:]:
        s = a.shape
        in_specs.append(pl.BlockSpec(s, lambda b, s=s: (0,) * len(s)))
    row = pl.BlockSpec((1, 1, D_), lambda b: (b, 0, 0))
    out = pl.pallas_call(
        _mixer_body,
        grid=(B_,),
        in_specs=in_specs,
        out_specs=[row, row, row, row],
        out_shape=[jax.ShapeDtypeStruct((B_, 1, D_), jnp.float32)] * 4,
    )(*args)
    return tuple(o.reshape(B_, D_) for o in out)


RNA_RB_ = 512
RNA_NRB_ = (NRNA_ + RNA_RB_ - 1) // RNA_RB_
_NPAD_ = RNA_NRB_ * RNA_RB_


_PART_ = N_ // 8
_NVR_ = _PART_ // 16
_NEG_ = -3.0e38
_BIGI_ = 2**31 - 1


def _sc_mesh():
    return plsc.VectorSubcoreMesh(core_axis_name="c", subcore_axis_name="s")


_SC_CP_ = pltpu.CompilerParams(needs_layout_passes=False)


def _sc_local_topk_body(logits_hbm, m_hbm, s_hbm, outv_hbm, outi_hbm, ap_hbm,
                        loc_v, ap_v, ms_v, topv_v, topi_v):
    c = lax.axis_index("c")
    s = lax.axis_index("s")
    b = c * 2 + s // 8
    part = s % 8
    base = part * _PART_
    pltpu.sync_copy(logits_hbm.at[b, pl.ds(base, _PART_)], loc_v)

    pltpu.sync_copy(m_hbm.at[b], ms_v)
    mvec = ms_v[...]
    pltpu.sync_copy(s_hbm.at[b], ms_v)
    rinv = 1.0 / ms_v[...]

    def ap_body(j, u):
        for q in range(4):
            off = pl.multiple_of(j * 64 + q * 16, 16)
            ap_v[pl.ds(off, 16)] = jnp.exp(loc_v[pl.ds(off, 16)] - mvec) * rinv
        return u

    lax.fori_loop(0, _NVR_ // 4, ap_body, 0)
    pltpu.sync_copy(ap_v, ap_hbm.at[b, pl.ds(base, _PART_)])

    iota = lax.iota(jnp.int32, 16)
    negs = jnp.full((16,), _NEG_, jnp.float32)
    bigs = jnp.full((16,), _BIGI_, jnp.int32)
    gbase = b * N_ + base

    def pass_body(t, carry):
        topv, topi = carry

        def scan_body(j, c2):
            accs = list(c2)
            for q in range(4):
                bv, bi = accs[q]
                off = pl.multiple_of(j * 64 + q * 16, 16)
                v = loc_v[pl.ds(off, 16)]
                idx = gbase + j * 64 + q * 16 + iota
                take = v > bv
                accs[q] = (jnp.where(take, v, bv), jnp.where(take, idx, bi))
            return tuple(accs)

        accs = lax.fori_loop(0, _NVR_ // 4, scan_body,
                             tuple((negs, bigs) for _ in range(4)))
        bv, bi = accs[0]
        for q in range(1, 4):
            v, idx = accs[q]
            take = (v > bv) | ((v == bv) & (idx < bi))
            bv = jnp.where(take, v, bv)
            bi = jnp.where(take, idx, bi)
        m = jnp.max(bv)
        wi = jnp.min(jnp.where(bv == m, bi, bigs))
        p = wi - gbase
        j0 = pl.multiple_of(jnp.bitwise_and(p, jnp.int32(~15)), 16)
        lane = jnp.bitwise_and(p, jnp.int32(15))
        v = loc_v[pl.ds(j0, 16)]
        loc_v[pl.ds(j0, 16)] = jnp.where(iota == lane, _NEG_, v)
        topv = jnp.where(iota == t, m, topv)
        topi = jnp.where(iota == t, wi, topi)
        return topv, topi

    topv, topi = lax.fori_loop(0, K_, pass_body,
                               (negs, jnp.zeros((16,), jnp.int32)))
    topv_v[...] = topv
    topi_v[...] = topi
    row = b * 8 + part
    pltpu.sync_copy(topv_v, outv_hbm.at[row])
    pltpu.sync_copy(topi_v, outi_hbm.at[row])


def _sc_local_topk(logits, m16, s16):
    call = pl.kernel(
        _sc_local_topk_body,
        out_type=[jax.ShapeDtypeStruct((B_ * 8, 16), jnp.float32),
                  jax.ShapeDtypeStruct((B_ * 8, 16), jnp.int32),
                  jax.ShapeDtypeStruct((B_, N_), jnp.float32)],
        mesh=_sc_mesh(),
        compiler_params=_SC_CP_,
        scratch_types=[
            pltpu.VMEM((_PART_,), jnp.float32),
            pltpu.VMEM((_PART_,), jnp.float32),
            pltpu.VMEM((16,), jnp.float32),
            pltpu.VMEM((16,), jnp.float32),
            pltpu.VMEM((16,), jnp.int32),
        ],
    )
    return call(logits, m16, s16)


def _sc_merge_gather_body(cv_hbm, ci_hbm, feats_hbm, out_hbm,
                          candv_v, candi_v, gidx_v, rows_v, sem):
    c = lax.axis_index("c")
    s = lax.axis_index("s")
    b = c * 2 + s // 8
    part = s % 8
    iota = lax.iota(jnp.int32, 16)
    negs = jnp.full((16,), _NEG_, jnp.float32)
    bigs = jnp.full((16,), _BIGI_, jnp.int32)

    @pl.when(part == 0)
    def _merge():
        b0 = pl.multiple_of(b * 8, 8)
        pltpu.sync_copy(cv_hbm.at[pl.ds(b0, 8)], candv_v)
        pltpu.sync_copy(ci_hbm.at[pl.ds(b0, 8)], candi_v)

        def mpass(t, gv):
            def mscan(r, c2):
                bv, bi = c2
                v = candv_v[r]
                i = candi_v[r]
                take = (v > bv) | ((v == bv) & (i < bi))
                return jnp.where(take, v, bv), jnp.where(take, i, bi)

            bv, bi = lax.fori_loop(0, 8, mscan, (negs, bigs))
            m = jnp.max(bv)
            wi = jnp.min(jnp.where(bv == m, bi, bigs))

            def mclear(r, u):
                v = candv_v[r]
                candv_v[r] = jnp.where(candi_v[r] == wi, _NEG_, v)
                return u

            lax.fori_loop(0, 8, mclear, 0)
            return jnp.where(iota == t, wi, gv)

        gv = lax.fori_loop(0, K_, mpass, jnp.zeros((16,), jnp.int32))
        gidx_v[...] = gv
        pltpu.async_copy(feats_hbm.at[gidx_v], rows_v, sem).wait()
        pltpu.sync_copy(rows_v, out_hbm.at[b])


def _sc_merge_gather(cand_v, cand_i, feats):
    call = pl.kernel(
        _sc_merge_gather_body,
        out_type=jax.ShapeDtypeStruct((B_, 16, D_), jnp.float32),
        mesh=_sc_mesh(),
        compiler_params=_SC_CP_,
        scratch_types=[
            pltpu.VMEM((8, 16), jnp.float32),
            pltpu.VMEM((8, 16), jnp.int32),
            pltpu.VMEM((16,), jnp.int32),
            pltpu.VMEM((16, D_), jnp.float32),
            pltpu.SemaphoreType.DMA,
        ],
    )
    return call(cand_v, cand_i, feats.reshape(B_ * N_, D_))[:, :K_, :]


def kernel(feats, feats_deep, rna_seq, params):
    P = params
    logits, m, s, bds, rna_emb, rna_proj = _fused_call(feats_deep, rna_seq, P)
    m16 = jnp.broadcast_to(m, (B_, 16))
    s16 = jnp.broadcast_to(s, (B_, 16))
    cand_v, cand_i, a_patch = _sc_local_topk(logits, m16, s16)
    topk_feats = _sc_merge_gather(cand_v, cand_i, feats)
    bsel, psum, a_feat, bdp = _mixer_call(topk_feats, bds, P)
    return (bsel, psum, bdp, bds, rna_proj, rna_emb, a_feat,
            a_patch.reshape(B_, N_, 1))
